# baseline (speedup 1.0000x reference)
import functools

import jax
import jax.numpy as jnp
from jax import lax
from jax.experimental import pallas as pl
from jax.experimental.pallas import tpu as pltpu
from jax.experimental.pallas import tpu_sc as plsc

N_SC = 2
N_SUB = 16
NW = N_SC * N_SUB
CHUNK = 2000


def _mesh():
    return plsc.VectorSubcoreMesh(core_axis_name="c", subcore_axis_name="s")


_SC_PARAMS = pltpu.CompilerParams(use_tc_tiling_on_sc=False)
_SC_PARAMS_NOLAYOUT = pltpu.CompilerParams(use_tc_tiling_on_sc=False,
                                           needs_layout_passes=False)


def _sc_degree_body(n_pad, epw, ei_hbm, ones_hbm, zeros_hbm, deg_out,
                    dst_v, ones_v, deg_sh, sem):
    c = lax.axis_index("c")
    s = lax.axis_index("s")
    wid = s * N_SC + c
    npc = n_pad // N_SUB
    d1 = pltpu.async_copy(zeros_hbm.at[pl.ds(s * npc, npc)],
                          deg_sh.at[pl.ds(s * npc, npc)], sem)
    d2 = pltpu.async_copy(ones_hbm, ones_v, sem)
    d3 = pltpu.async_copy(ei_hbm.at[1, pl.ds(wid * epw, epw)], dst_v, sem)
    d1.wait()
    d2.wait()
    d3.wait()
    plsc.subcore_barrier()
    pltpu.sync_copy(ones_v, deg_sh.at[dst_v], add=True)
    plsc.subcore_barrier()
    pltpu.sync_copy(deg_sh.at[pl.ds(s * npc, npc)],
                    deg_out.at[c, pl.ds(s * npc, npc)])


def _sc_degree(ei, n_pad):
    e = ei.shape[1]
    epw = e // NW
    ones = jnp.ones((epw,), jnp.float32)
    zeros = jnp.zeros((n_pad,), jnp.float32)
    return pl.kernel(
        functools.partial(_sc_degree_body, n_pad, epw),
        mesh=_mesh(),
        compiler_params=_SC_PARAMS,
        out_type=jax.ShapeDtypeStruct((N_SC, n_pad), jnp.float32),
        scratch_types=[
            pltpu.VMEM((epw,), jnp.int32),
            pltpu.VMEM((epw,), jnp.float32),
            pltpu.VMEM_SHARED((n_pad,), jnp.float32),
            pltpu.SemaphoreType.DMA,
        ],
    )(ei, ones, zeros)


def _tc_mm_body(x_ref, w1_ref, h_ref):
    h_ref[...] = jnp.dot(x_ref[...], w1_ref[...],
                         preferred_element_type=jnp.float32)


def _tc_mm(x, w1):
    n = x.shape[0]
    dh = w1.shape[1]
    return pl.pallas_call(
        _tc_mm_body,
        out_shape=jax.ShapeDtypeStruct((n, dh), jnp.float32),
    )(x, w1)


def _tc_dinv_body(degp_ref, dinv_ref):
    deg = degp_ref[0:1, :] + degp_ref[1:2, :] + 1.0
    dinv_ref[...] = lax.rsqrt(deg)


def _tc_dinv(degp):
    n_pad = degp.shape[1]
    return pl.pallas_call(
        _tc_dinv_body,
        out_shape=jax.ShapeDtypeStruct((1, n_pad), jnp.float32),
    )(degp)


def _sc_rows_body(n_pad, n, epw, d, ei_hbm, h_hbm, dinv_hbm, zeros_hbm,
                  acc_out, h1p_out, src_v, dst_v2, rows0_v, rows1_v, nrow_v,
                  dinv_v, acc_sh, sem, sg0, sg1):
    c = lax.axis_index("c")
    s = lax.axis_index("s")
    wid = s * N_SC + c
    npc = n_pad // N_SUB
    nbase = s * npc
    nchunks = epw // CHUNK
    tail = n - (N_SUB - 1) * npc
    pre = [pltpu.async_copy(zeros_hbm.at[pl.ds(nbase, npc)],
                            acc_sh.at[pl.ds(nbase, npc)], sem),
           pltpu.async_copy(ei_hbm.at[0, pl.ds(wid * epw, epw)], src_v, sem)]
    for k in range(nchunks):
        pre.append(pltpu.async_copy(
            ei_hbm.at[1, pl.ds(wid * epw + k * CHUNK, CHUNK)],
            dst_v2.at[k], sem))
    pltpu.sync_copy(dinv_hbm.at[0, pl.ds(nbase, npc)], dinv_v)

    def scale_rows(nvalid):
        pltpu.sync_copy(h_hbm.at[pl.ds(nbase, nvalid)],
                        nrow_v.at[pl.ds(0, nvalid)])

        @plsc.parallel_loop(0, nvalid // 16, unroll=2)
        def _(g):
            rows = lax.iota(jnp.int32, 16) + g * 16
            dvec = dinv_v[pl.ds(g * 16, 16)]
            for j in range(d):
                cj = jnp.full((16,), j, jnp.int32)
                v = plsc.load_gather(nrow_v, [rows, cj]) * dvec
                plsc.store_scatter(nrow_v, [rows, cj], v)
        pltpu.sync_copy(nrow_v.at[pl.ds(0, nvalid)],
                        h1p_out.at[pl.ds(nbase, nvalid)])

    @pl.when(s == N_SUB - 1)
    def _():
        scale_rows(tail)

    @pl.when(s < N_SUB - 1)
    def _():
        scale_rows(npc)

    for p in pre:
        p.wait()
    plsc.subcore_barrier()
    rows = [rows0_v, rows1_v]
    sg = [sg0, sg1]

    def start_gather(k):
        return pltpu.async_copy(
            h1p_out.at[src_v.at[pl.ds(k * CHUNK, CHUNK)]],
            rows[k % 2], sg[k % 2])

    g = start_gather(0)
    for k in range(nchunks):
        g.wait()
        if k + 1 < nchunks:
            g = start_gather(k + 1)
        pltpu.sync_copy(rows[k % 2], acc_sh.at[dst_v2.at[k]], add=True)
    plsc.subcore_barrier()
    pltpu.sync_copy(acc_sh.at[pl.ds(nbase, npc)],
                    acc_out.at[c, pl.ds(nbase, npc)])


def _sc_rows(ei, h, dinv_row, n_pad):
    e = ei.shape[1]
    epw = e // NW
    n, d = h.shape
    zeros = jnp.zeros((n_pad, d), jnp.float32)
    return pl.kernel(
        functools.partial(_sc_rows_body, n_pad, n, epw, d),
        mesh=_mesh(),
        compiler_params=_SC_PARAMS_NOLAYOUT,
        out_type=[
            jax.ShapeDtypeStruct((N_SC, n_pad, d), jnp.float32),
            jax.ShapeDtypeStruct((n_pad, d), jnp.float32),
        ],
        scratch_types=[
            pltpu.VMEM((epw,), jnp.int32),
            pltpu.VMEM((epw // CHUNK, CHUNK), jnp.int32),
            pltpu.VMEM((CHUNK, d), jnp.float32),
            pltpu.VMEM((CHUNK, d), jnp.float32),
            pltpu.VMEM((n_pad // N_SUB, d), jnp.float32),
            pltpu.VMEM((n_pad // N_SUB,), jnp.float32),
            pltpu.VMEM_SHARED((n_pad, d), jnp.float32),
            pltpu.SemaphoreType.DMA,
            pltpu.SemaphoreType.DMA,
            pltpu.SemaphoreType.DMA,
        ],
    )(ei, h, dinv_row, zeros)


def _sc_scalar_body(n_pad, epw, d, ei_hbm, accp_hbm, h1p_hbm, dinv_hbm,
                    b1_hbm, w2_hbm, zeros_hbm, acc_out, gp_out, src_v,
                    dst_v2, vals0_v, vals1_v, gp_all_v, a0_v, a1_v, h1_v,
                    dinv_v, b1_v, w2_v, gps_v, acc_sh, gp_sh, sem, ss0, ss1):
    c = lax.axis_index("c")
    s = lax.axis_index("s")
    wid = s * N_SC + c
    npc = n_pad // N_SUB
    nbase = s * npc
    nchunks = epw // CHUNK
    pre = [pltpu.async_copy(zeros_hbm.at[pl.ds(nbase, npc)],
                           acc_sh.at[pl.ds(nbase, npc)], sem),
           pltpu.async_copy(ei_hbm.at[0, pl.ds(wid * epw, epw)], src_v, sem)]
    for k in range(nchunks):
        pre.append(pltpu.async_copy(
            ei_hbm.at[1, pl.ds(wid * epw + k * CHUNK, CHUNK)],
            dst_v2.at[k], sem))
    pltpu.sync_copy(accp_hbm.at[0, pl.ds(nbase, npc)], a0_v)
    pltpu.sync_copy(accp_hbm.at[1, pl.ds(nbase, npc)], a1_v)
    pltpu.sync_copy(h1p_hbm.at[pl.ds(nbase, npc)], h1_v)
    pltpu.sync_copy(dinv_hbm.at[0, pl.ds(nbase, npc)], dinv_v)
    pltpu.sync_copy(b1_hbm, b1_v)
    pltpu.sync_copy(w2_hbm, w2_v)

    b1 = b1_v[...]
    w2 = w2_v[...]

    @plsc.parallel_loop(0, npc // 16, unroll=2)
    def _(g):
        rows = lax.iota(jnp.int32, 16) + g * 16
        dvec = dinv_v[pl.ds(g * 16, 16)]
        acc = jnp.zeros((16,), jnp.float32)
        for j in range(d):
            cj = jnp.full((16,), j, jnp.int32)
            v = (plsc.load_gather(a0_v, [rows, cj])
                 + plsc.load_gather(a1_v, [rows, cj])
                 + plsc.load_gather(h1_v, [rows, cj]))
            out1 = v * dvec + b1[j]
            acc = acc + jnp.maximum(out1, 0.0) * w2[j]
        gps_v[pl.ds(g * 16, 16)] = dvec * acc
    pltpu.sync_copy(gps_v, gp_sh.at[pl.ds(nbase, npc)])

    @pl.when(c == 0)
    def _():
        pltpu.sync_copy(gps_v, gp_out.at[pl.ds(nbase, npc)])

    for p in pre:
        p.wait()
    plsc.subcore_barrier()
    pltpu.sync_copy(gp_sh, gp_all_v)
    vals = [vals0_v, vals1_v]
    ss = [ss0, ss1]
    sd = [None] * nchunks
    for k in range(nchunks):
        if k >= 2:
            sd[k - 2].wait()
        vk = vals[k % 2]

        @plsc.parallel_loop(0, CHUNK // 16, unroll=4)
        def _(j):
            idx = src_v[pl.ds(k * CHUNK + j * 16, 16)]
            vk[pl.ds(j * 16, 16)] = plsc.load_gather(gp_all_v, [idx])
        sd[k] = pltpu.async_copy(vk, acc_sh.at[dst_v2.at[k]], ss[k % 2],
                                 add=True)
    sd[nchunks - 2].wait()
    sd[nchunks - 1].wait()
    plsc.subcore_barrier()
    pltpu.sync_copy(acc_sh.at[pl.ds(nbase, npc)],
                    acc_out.at[c, pl.ds(nbase, npc)])


def _sc_scalar(ei, accp, h1p, dinv_row, b1, w2, n_pad):
    e = ei.shape[1]
    epw = e // NW
    d = h1p.shape[1]
    zeros = jnp.zeros((n_pad,), jnp.float32)
    npc = n_pad // N_SUB
    return pl.kernel(
        functools.partial(_sc_scalar_body, n_pad, epw, d),
        mesh=_mesh(),
        compiler_params=_SC_PARAMS_NOLAYOUT,
        out_type=[
            jax.ShapeDtypeStruct((N_SC, n_pad), jnp.float32),
            jax.ShapeDtypeStruct((n_pad,), jnp.float32),
        ],
        scratch_types=[
            pltpu.VMEM((epw,), jnp.int32),
            pltpu.VMEM((epw // CHUNK, CHUNK), jnp.int32),
            pltpu.VMEM((CHUNK,), jnp.float32),
            pltpu.VMEM((CHUNK,), jnp.float32),
            pltpu.VMEM((n_pad,), jnp.float32),
            pltpu.VMEM((npc, d), jnp.float32),
            pltpu.VMEM((npc, d), jnp.float32),
            pltpu.VMEM((npc, d), jnp.float32),
            pltpu.VMEM((npc,), jnp.float32),
            pltpu.VMEM((d,), jnp.float32),
            pltpu.VMEM((d,), jnp.float32),
            pltpu.VMEM((npc,), jnp.float32),
            pltpu.VMEM_SHARED((n_pad,), jnp.float32),
            pltpu.VMEM_SHARED((n_pad,), jnp.float32),
            pltpu.SemaphoreType.DMA,
            pltpu.SemaphoreType.DMA,
            pltpu.SemaphoreType.DMA,
        ],
    )(ei, accp, h1p, dinv_row, b1, w2, zeros)


def _tc_final_body(accp_ref, gp_ref, dinv_ref, b2_ref, out_ref):
    s2 = accp_ref[0:1, :] + accp_ref[1:2, :] + gp_ref[...]
    out_ref[...] = dinv_ref[...] * s2 + b2_ref[...]


def _tc_final(accp, gp_row, dinv_row, b2):
    n_pad = gp_row.shape[1]
    return pl.pallas_call(
        _tc_final_body,
        out_shape=jax.ShapeDtypeStruct((1, n_pad), jnp.float32),
    )(accp, gp_row, dinv_row, b2)


def kernel(x, edge_index, W1, b1, W2, b2):
    n = x.shape[0]
    n_pad = ((n + NW * 8 - 1) // (NW * 8)) * (NW * 8)
    ei = edge_index.astype(jnp.int32)

    degp = _sc_degree(ei, n_pad)

    h = _tc_mm(x, W1)
    dinv_row = _tc_dinv(degp)

    acc1, h1p = _sc_rows(ei, h, dinv_row, n_pad)

    acc2, gp = _sc_scalar(ei, acc1, h1p, dinv_row, b1, W2.reshape(-1), n_pad)

    out_row = _tc_final(acc2, gp.reshape(1, n_pad), dinv_row,
                        b2.reshape(1, 1))
    return out_row[0, :n].reshape(n, 1)


# speedup vs baseline: 1.0901x; 1.0901x over previous
import functools

import jax
import jax.numpy as jnp
from jax import lax
from jax.experimental import pallas as pl
from jax.experimental.pallas import tpu as pltpu
from jax.experimental.pallas import tpu_sc as plsc

N_SC = 2
N_SUB = 16
NW = N_SC * N_SUB
CHUNK = 2000


def _mesh():
    return plsc.VectorSubcoreMesh(core_axis_name="c", subcore_axis_name="s")


_SC_PARAMS = pltpu.CompilerParams(use_tc_tiling_on_sc=False)
_SC_PARAMS_NOLAYOUT = pltpu.CompilerParams(use_tc_tiling_on_sc=False,
                                           needs_layout_passes=False)


def _sc_degree_body(n_pad, epw, ei_hbm, ones_hbm, zeros_hbm, deg_out,
                    dst_v, ones_v, deg_sh, sem):
    c = lax.axis_index("c")
    s = lax.axis_index("s")
    wid = s * N_SC + c
    npc = n_pad // N_SUB
    d1 = pltpu.async_copy(zeros_hbm.at[pl.ds(s * npc, npc)],
                          deg_sh.at[pl.ds(s * npc, npc)], sem)
    d2 = pltpu.async_copy(ones_hbm, ones_v, sem)
    d3 = pltpu.async_copy(ei_hbm.at[1, pl.ds(wid * epw, epw)], dst_v, sem)
    d1.wait()
    d2.wait()
    d3.wait()
    plsc.subcore_barrier()
    pltpu.sync_copy(ones_v, deg_sh.at[dst_v], add=True)
    plsc.subcore_barrier()
    pltpu.sync_copy(deg_sh.at[pl.ds(s * npc, npc)],
                    deg_out.at[c, pl.ds(s * npc, npc)])


def _sc_degree(ei, n_pad):
    e = ei.shape[1]
    epw = e // NW
    ones = jnp.ones((epw,), jnp.float32)
    zeros = jnp.zeros((n_pad,), jnp.float32)
    return pl.kernel(
        functools.partial(_sc_degree_body, n_pad, epw),
        mesh=_mesh(),
        compiler_params=_SC_PARAMS,
        out_type=jax.ShapeDtypeStruct((N_SC, n_pad), jnp.float32),
        scratch_types=[
            pltpu.VMEM((epw,), jnp.int32),
            pltpu.VMEM((epw,), jnp.float32),
            pltpu.VMEM_SHARED((n_pad,), jnp.float32),
            pltpu.SemaphoreType.DMA,
        ],
    )(ei, ones, zeros)


def _tc_mm_body(x_ref, w1_ref, ht_ref):
    ht_ref[...] = lax.dot_general(
        w1_ref[...], x_ref[...], (((0,), (1,)), ((), ())),
        preferred_element_type=jnp.float32)


def _tc_mm(x, w1):
    n = x.shape[0]
    dh = w1.shape[1]
    return pl.pallas_call(
        _tc_mm_body,
        out_shape=jax.ShapeDtypeStruct((dh, n), jnp.float32),
    )(x, w1)


def _tc_dinv_body(degp_ref, dinv_ref):
    deg = degp_ref[0:1, :] + degp_ref[1:2, :] + 1.0
    dinv_ref[...] = lax.rsqrt(deg)


def _tc_dinv(degp):
    n_pad = degp.shape[1]
    return pl.pallas_call(
        _tc_dinv_body,
        out_shape=jax.ShapeDtypeStruct((1, n_pad), jnp.float32),
    )(degp)


def _sc_rows_body(n_pad, n, epw, d, ei_hbm, h_hbm, dinv_hbm, zeros_hbm,
                  acc_out, h1p_out, src_v, dst_v2, rows0_v, rows1_v, nrow_v,
                  ht_v, dinv_v, acc_sh, h1p_sh, sem, sg0, sg1):
    c = lax.axis_index("c")
    s = lax.axis_index("s")
    wid = s * N_SC + c
    npc = n_pad // N_SUB
    nbase = s * npc
    nchunks = epw // CHUNK
    tail = n - (N_SUB - 1) * npc
    pre = [pltpu.async_copy(zeros_hbm.at[pl.ds(nbase, npc)],
                            acc_sh.at[pl.ds(nbase, npc)], sem),
           pltpu.async_copy(ei_hbm.at[0, pl.ds(wid * epw, epw)], src_v, sem)]
    for k in range(nchunks):
        pre.append(pltpu.async_copy(
            ei_hbm.at[1, pl.ds(wid * epw + k * CHUNK, CHUNK)],
            dst_v2.at[k], sem))
    pltpu.sync_copy(dinv_hbm.at[0, pl.ds(nbase, npc)], dinv_v)

    def scale_rows(nvalid):
        pltpu.sync_copy(h_hbm.at[:, pl.ds(nbase, nvalid)],
                        ht_v.at[:, pl.ds(0, nvalid)])

        @plsc.parallel_loop(0, nvalid // 16, unroll=2)
        def _(g):
            rows = lax.iota(jnp.int32, 16) + g * 16
            dvec = dinv_v[pl.ds(g * 16, 16)]
            for j in range(d):
                cj = jnp.full((16,), j, jnp.int32)
                v = ht_v[j, pl.ds(g * 16, 16)] * dvec
                plsc.store_scatter(nrow_v, [rows, cj], v)
        pltpu.sync_copy(nrow_v.at[pl.ds(0, nvalid)],
                        h1p_sh.at[pl.ds(nbase, nvalid)])

        @pl.when(c == 0)
        def _():
            pltpu.sync_copy(nrow_v.at[pl.ds(0, nvalid)],
                            h1p_out.at[pl.ds(nbase, nvalid)])

    @pl.when(s == N_SUB - 1)
    def _():
        scale_rows(tail)

    @pl.when(s < N_SUB - 1)
    def _():
        scale_rows(npc)

    for p in pre:
        p.wait()
    plsc.subcore_barrier()
    rows = [rows0_v, rows1_v]
    sg = [sg0, sg1]

    def start_gather(k):
        return pltpu.async_copy(
            h1p_sh.at[src_v.at[pl.ds(k * CHUNK, CHUNK)]],
            rows[k % 2], sg[k % 2])

    g = start_gather(0)
    for k in range(nchunks):
        g.wait()
        if k + 1 < nchunks:
            g = start_gather(k + 1)
        pltpu.sync_copy(rows[k % 2], acc_sh.at[dst_v2.at[k]], add=True)
    plsc.subcore_barrier()
    pltpu.sync_copy(acc_sh.at[pl.ds(nbase, npc)],
                    acc_out.at[c, pl.ds(nbase, npc)])


def _sc_rows(ei, h, dinv_row, n_pad):
    e = ei.shape[1]
    epw = e // NW
    d, n = h.shape
    zeros = jnp.zeros((n_pad, d), jnp.float32)
    return pl.kernel(
        functools.partial(_sc_rows_body, n_pad, n, epw, d),
        mesh=_mesh(),
        compiler_params=_SC_PARAMS_NOLAYOUT,
        out_type=[
            jax.ShapeDtypeStruct((N_SC, n_pad, d), jnp.float32),
            jax.ShapeDtypeStruct((n_pad, d), jnp.float32),
        ],
        scratch_types=[
            pltpu.VMEM((epw,), jnp.int32),
            pltpu.VMEM((epw // CHUNK, CHUNK), jnp.int32),
            pltpu.VMEM((CHUNK, d), jnp.float32),
            pltpu.VMEM((CHUNK, d), jnp.float32),
            pltpu.VMEM((n_pad // N_SUB, d), jnp.float32),
            pltpu.VMEM((d, n_pad // N_SUB), jnp.float32),
            pltpu.VMEM((n_pad // N_SUB,), jnp.float32),
            pltpu.VMEM_SHARED((n_pad, d), jnp.float32),
            pltpu.VMEM_SHARED((n_pad, d), jnp.float32),
            pltpu.SemaphoreType.DMA,
            pltpu.SemaphoreType.DMA,
            pltpu.SemaphoreType.DMA,
        ],
    )(ei, h, dinv_row, zeros)


def _sc_scalar_body(n_pad, epw, d, ei_hbm, accp_hbm, h1p_hbm, dinv_hbm,
                    b1_hbm, w2_hbm, zeros_hbm, acc_out, gp_out, src_v,
                    dst_v2, vals0_v, vals1_v, gp_all_v, a0_v, a1_v, h1_v,
                    dinv_v, b1_v, w2_v, gps_v, acc_sh, gp_sh, sem, ss0, ss1):
    c = lax.axis_index("c")
    s = lax.axis_index("s")
    wid = s * N_SC + c
    npc = n_pad // N_SUB
    nbase = s * npc
    nchunks = epw // CHUNK
    pre = [pltpu.async_copy(zeros_hbm.at[pl.ds(nbase, npc)],
                           acc_sh.at[pl.ds(nbase, npc)], sem),
           pltpu.async_copy(ei_hbm.at[0, pl.ds(wid * epw, epw)], src_v, sem)]
    for k in range(nchunks):
        pre.append(pltpu.async_copy(
            ei_hbm.at[1, pl.ds(wid * epw + k * CHUNK, CHUNK)],
            dst_v2.at[k], sem))
    pltpu.sync_copy(accp_hbm.at[0, pl.ds(nbase, npc)], a0_v)
    pltpu.sync_copy(accp_hbm.at[1, pl.ds(nbase, npc)], a1_v)
    pltpu.sync_copy(h1p_hbm.at[pl.ds(nbase, npc)], h1_v)
    pltpu.sync_copy(dinv_hbm.at[0, pl.ds(nbase, npc)], dinv_v)
    pltpu.sync_copy(b1_hbm, b1_v)
    pltpu.sync_copy(w2_hbm, w2_v)

    b1 = b1_v[...]
    w2 = w2_v[...]

    @plsc.parallel_loop(0, npc // 16, unroll=2)
    def _(g):
        rows = lax.iota(jnp.int32, 16) + g * 16
        dvec = dinv_v[pl.ds(g * 16, 16)]
        acc = jnp.zeros((16,), jnp.float32)
        for j in range(d):
            cj = jnp.full((16,), j, jnp.int32)
            v = (plsc.load_gather(a0_v, [rows, cj])
                 + plsc.load_gather(a1_v, [rows, cj])
                 + plsc.load_gather(h1_v, [rows, cj]))
            out1 = v * dvec + b1[j]
            acc = acc + jnp.maximum(out1, 0.0) * w2[j]
        gps_v[pl.ds(g * 16, 16)] = dvec * acc
    pltpu.sync_copy(gps_v, gp_sh.at[pl.ds(nbase, npc)])

    @pl.when(c == 0)
    def _():
        pltpu.sync_copy(gps_v, gp_out.at[pl.ds(nbase, npc)])

    for p in pre:
        p.wait()
    plsc.subcore_barrier()
    pltpu.sync_copy(gp_sh, gp_all_v)
    vals = [vals0_v, vals1_v]
    ss = [ss0, ss1]
    sd = [None] * nchunks
    for k in range(nchunks):
        if k >= 2:
            sd[k - 2].wait()
        vk = vals[k % 2]

        @plsc.parallel_loop(0, CHUNK // 16, unroll=4)
        def _(j):
            idx = src_v[pl.ds(k * CHUNK + j * 16, 16)]
            vk[pl.ds(j * 16, 16)] = plsc.load_gather(gp_all_v, [idx])
        sd[k] = pltpu.async_copy(vk, acc_sh.at[dst_v2.at[k]], ss[k % 2],
                                 add=True)
    sd[nchunks - 2].wait()
    sd[nchunks - 1].wait()
    plsc.subcore_barrier()
    pltpu.sync_copy(acc_sh.at[pl.ds(nbase, npc)],
                    acc_out.at[c, pl.ds(nbase, npc)])


def _sc_scalar(ei, accp, h1p, dinv_row, b1, w2, n_pad):
    e = ei.shape[1]
    epw = e // NW
    d = h1p.shape[1]
    zeros = jnp.zeros((n_pad,), jnp.float32)
    npc = n_pad // N_SUB
    return pl.kernel(
        functools.partial(_sc_scalar_body, n_pad, epw, d),
        mesh=_mesh(),
        compiler_params=_SC_PARAMS_NOLAYOUT,
        out_type=[
            jax.ShapeDtypeStruct((N_SC, n_pad), jnp.float32),
            jax.ShapeDtypeStruct((n_pad,), jnp.float32),
        ],
        scratch_types=[
            pltpu.VMEM((epw,), jnp.int32),
            pltpu.VMEM((epw // CHUNK, CHUNK), jnp.int32),
            pltpu.VMEM((CHUNK,), jnp.float32),
            pltpu.VMEM((CHUNK,), jnp.float32),
            pltpu.VMEM((n_pad,), jnp.float32),
            pltpu.VMEM((npc, d), jnp.float32),
            pltpu.VMEM((npc, d), jnp.float32),
            pltpu.VMEM((npc, d), jnp.float32),
            pltpu.VMEM((npc,), jnp.float32),
            pltpu.VMEM((d,), jnp.float32),
            pltpu.VMEM((d,), jnp.float32),
            pltpu.VMEM((npc,), jnp.float32),
            pltpu.VMEM_SHARED((n_pad,), jnp.float32),
            pltpu.VMEM_SHARED((n_pad,), jnp.float32),
            pltpu.SemaphoreType.DMA,
            pltpu.SemaphoreType.DMA,
            pltpu.SemaphoreType.DMA,
        ],
    )(ei, accp, h1p, dinv_row, b1, w2, zeros)


def _tc_final_body(accp_ref, gp_ref, dinv_ref, b2_ref, out_ref):
    s2 = accp_ref[0:1, :] + accp_ref[1:2, :] + gp_ref[...]
    out_ref[...] = dinv_ref[...] * s2 + b2_ref[...]


def _tc_final(accp, gp_row, dinv_row, b2):
    n_pad = gp_row.shape[1]
    return pl.pallas_call(
        _tc_final_body,
        out_shape=jax.ShapeDtypeStruct((1, n_pad), jnp.float32),
    )(accp, gp_row, dinv_row, b2)


def kernel(x, edge_index, W1, b1, W2, b2):
    n = x.shape[0]
    n_pad = ((n + NW * 8 - 1) // (NW * 8)) * (NW * 8)
    ei = edge_index.astype(jnp.int32)

    degp = _sc_degree(ei, n_pad)

    h = _tc_mm(x, W1)
    dinv_row = _tc_dinv(degp)

    acc1, h1p = _sc_rows(ei, h, dinv_row, n_pad)

    acc2, gp = _sc_scalar(ei, acc1, h1p, dinv_row, b1, W2.reshape(-1), n_pad)

    out_row = _tc_final(acc2, gp.reshape(1, n_pad), dinv_row,
                        b2.reshape(1, 1))
    return out_row[0, :n].reshape(n, 1)
